# Initial kernel scaffold; baseline (speedup 1.0000x reference)
import jax
import jax.numpy as jnp
from jax.experimental import pallas as pl


def kernel(x, edge_list, edge_attr, W1, b1, Wfc, bfc):
    raise NotImplementedError("write your pallas kernel here")


# speedup vs baseline: 73.8915x; 73.8915x over previous
import functools

import jax
import jax.numpy as jnp
from jax import lax
from jax.experimental import pallas as pl
from jax.experimental.pallas import tpu as pltpu
from jax.experimental.pallas import tpu_sc as plsc

N = 10000
E = 320000
L = 16
NC = 2
NS = 16
NW = NC * NS
R = N // L
R2 = 2 * N // L
EPW = E // NW
IT = EPW // L

_sc_mesh = plsc.VectorSubcoreMesh(
    core_axis_name="c", subcore_axis_name="s", num_cores=NC, num_subcores=NS)


def _sc_deg_body(dst_hbm, ew_hbm, degp_hbm, dstv, ewv, deg):
  wid = lax.axis_index("c") * NS + lax.axis_index("s")

  def zero(i, _):
    deg[pl.ds(i * L, L)] = jnp.zeros((L,), jnp.float32)
    return 0
  lax.fori_loop(0, R, zero, 0)

  off = wid * EPW
  pltpu.sync_copy(dst_hbm.at[pl.ds(off, EPW)], dstv)
  pltpu.sync_copy(ew_hbm.at[pl.ds(off, EPW)], ewv)

  def body(i, _):
    d = dstv[pl.ds(i * L, L)]
    w = ewv[pl.ds(i * L, L)]
    plsc.addupdate_scatter(deg, [d], w)
    return 0
  lax.fori_loop(0, IT, body, 0)

  pltpu.sync_copy(deg, degp_hbm.at[wid])


@functools.partial(
    pl.kernel,
    out_type=jax.ShapeDtypeStruct((NW, N), jnp.float32),
    mesh=_sc_mesh,
    scratch_types=[
        pltpu.VMEM((EPW,), jnp.int32),
        pltpu.VMEM((EPW,), jnp.float32),
        pltpu.VMEM((N,), jnp.float32),
    ],
    compiler_params=pltpu.CompilerParams(needs_layout_passes=False),
)
def _sc_deg(dst_hbm, ew_hbm, degp_hbm, dstv, ewv, deg):
  _sc_deg_body(dst_hbm, ew_hbm, degp_hbm, dstv, ewv, deg)


def _tc_prep_body(x_ref, w1_ref, degp_ref, h_ref, dinv_ref):
  h_ref[...] = jnp.dot(x_ref[...], w1_ref[...],
                       preferred_element_type=jnp.float32)
  deg = jnp.sum(degp_ref[...], axis=0) + 1.0
  dinv_ref[...] = lax.rsqrt(deg)


def _tc_prep(x, w1, degp):
  return pl.pallas_call(
      _tc_prep_body,
      out_shape=(
          jax.ShapeDtypeStruct((N, 2), jnp.float32),
          jax.ShapeDtypeStruct((80, 125), jnp.float32),
      ),
  )(x, w1, degp)


def _sc_agg_body(src_hbm, dst_hbm, ew_hbm, h_hbm, dv2_hbm, accp_hbm,
                 srcv, dstv, ewv, hd, dv2, acc):
  wid = lax.axis_index("c") * NS + lax.axis_index("s")

  pltpu.sync_copy(h_hbm, hd)
  pltpu.sync_copy(dv2_hbm, dv2)

  def scale(i, _):
    sl = pl.ds(i * L, L)
    hd[sl] = hd[sl] * dv2[sl]
    acc[sl] = jnp.zeros((L,), jnp.float32)
    return 0
  lax.fori_loop(0, R2, scale, 0)

  off = wid * EPW
  pltpu.sync_copy(src_hbm.at[pl.ds(off, EPW)], srcv)
  pltpu.sync_copy(dst_hbm.at[pl.ds(off, EPW)], dstv)
  pltpu.sync_copy(ew_hbm.at[pl.ds(off, EPW)], ewv)

  def body(i, _):
    s = srcv[pl.ds(i * L, L)]
    d = dstv[pl.ds(i * L, L)]
    w = ewv[pl.ds(i * L, L)]
    s2 = s << 1
    d2 = d << 1
    m0 = plsc.load_gather(hd, [s2]) * w
    m1 = plsc.load_gather(hd, [s2 + 1]) * w
    plsc.addupdate_scatter(acc, [d2], m0)
    plsc.addupdate_scatter(acc, [d2 + 1], m1)
    return 0
  lax.fori_loop(0, IT, body, 0)

  pltpu.sync_copy(acc, accp_hbm.at[wid])


@functools.partial(
    pl.kernel,
    out_type=jax.ShapeDtypeStruct((NW, 2 * N), jnp.float32),
    mesh=_sc_mesh,
    scratch_types=[
        pltpu.VMEM((EPW,), jnp.int32),
        pltpu.VMEM((EPW,), jnp.int32),
        pltpu.VMEM((EPW,), jnp.float32),
        pltpu.VMEM((2 * N,), jnp.float32),
        pltpu.VMEM((2 * N,), jnp.float32),
        pltpu.VMEM((2 * N,), jnp.float32),
    ],
    compiler_params=pltpu.CompilerParams(needs_layout_passes=False),
)
def _sc_agg(src_hbm, dst_hbm, ew_hbm, h_hbm, dv2_hbm, accp_hbm,
            srcv, dstv, ewv, hd, dv2, acc):
  _sc_agg_body(src_hbm, dst_hbm, ew_hbm, h_hbm, dv2_hbm, accp_hbm,
               srcv, dstv, ewv, hd, dv2, acc)


def _tc_final_body(accp_ref, hf_ref, dv2_ref, b1i_ref, wfc_ref, bfc_ref,
                   o_ref):
  acc = jnp.sum(accp_ref[...], axis=0)
  dv2 = dv2_ref[...]
  hf = hf_ref[...]
  out = dv2 * (acc + hf * dv2) + b1i_ref[...]
  out = jnp.maximum(out, 0.0)
  logit = jnp.sum(out * wfc_ref[...]) + bfc_ref[0]
  o_ref[0, 0] = 1.0 / (1.0 + jnp.exp(-logit))


def _tc_final(accp, hf, dv2, b1i, wfc, bfc):
  return pl.pallas_call(
      _tc_final_body,
      in_specs=[
          pl.BlockSpec(memory_space=pltpu.VMEM),
          pl.BlockSpec(memory_space=pltpu.VMEM),
          pl.BlockSpec(memory_space=pltpu.VMEM),
          pl.BlockSpec(memory_space=pltpu.VMEM),
          pl.BlockSpec(memory_space=pltpu.VMEM),
          pl.BlockSpec(memory_space=pltpu.SMEM),
      ],
      out_specs=pl.BlockSpec(memory_space=pltpu.SMEM),
      out_shape=jax.ShapeDtypeStruct((1, 1), jnp.float32),
  )(accp, hf, dv2, b1i, wfc, bfc)


def kernel(x, edge_list, edge_attr, W1, b1, Wfc, bfc):
  src = edge_list[0]
  dst = edge_list[1]
  ew = edge_attr.astype(jnp.float32)

  degp = _sc_deg(dst, ew)
  h, dinv = _tc_prep(x, W1, degp.reshape(NW, 80, 125))

  hf = h.reshape(2 * N)
  dv2 = jnp.repeat(dinv.reshape(N), 2)

  accp = _sc_agg(src, dst, ew, hf, dv2)

  out = _tc_final(
      accp.reshape(NW, 125, 160),
      hf.reshape(125, 160),
      dv2.reshape(125, 160),
      jnp.tile(b1, N).reshape(125, 160),
      Wfc.reshape(125, 160),
      bfc,
  )
  return out.reshape(())


# speedup vs baseline: 89.4672x; 1.2108x over previous
import functools

import jax
import jax.numpy as jnp
from jax import lax
from jax.experimental import pallas as pl
from jax.experimental.pallas import tpu as pltpu
from jax.experimental.pallas import tpu_sc as plsc

N = 10000
E = 320000
L = 16
NC = 2
NS = 16
NW = NC * NS
NP = 10240
STR = NP // NS
E1 = E // NS
E2 = E // NW

_sc_mesh = plsc.VectorSubcoreMesh(
    core_axis_name="c", subcore_axis_name="s", num_cores=NC, num_subcores=NS)


def _rsqrt16(v):
  i = plsc.bitcast(v, jnp.int32)
  y = plsc.bitcast(jnp.int32(0x5F3759DF) - (i >> 1), jnp.float32)
  for _ in range(3):
    y = y * (1.5 - 0.5 * v * y * y)
  return y


def _sc_main_body(src_hbm, dst_hbm, ew_hbm, hp_hbm,
                  accp0_hbm, accp1_hbm, dinv_hbm,
                  dstv, ewv, srcv, deg, sbuf, dbuf, h0d, h1d, acc0, acc1,
                  degparts, dinv_sh):
  cid = lax.axis_index("c")
  sid = lax.axis_index("s")
  wid = cid * NS + sid

  pltpu.sync_copy(dst_hbm.at[pl.ds(sid * E1, E1)], dstv)
  pltpu.sync_copy(ew_hbm.at[pl.ds(sid * E1, E1)], ewv)

  def zero_deg(i, _):
    deg[pl.ds(i * L, L)] = jnp.zeros((L,), jnp.float32)
    return 0
  lax.fori_loop(0, NP // L, zero_deg, 0)

  def p1(i, _):
    d = dstv[pl.ds(i * L, L)]
    w = ewv[pl.ds(i * L, L)]
    plsc.addupdate_scatter(deg, [d], w)
    return 0
  lax.fori_loop(0, E1 // L, p1, 0)

  pltpu.sync_copy(deg, degparts.at[pl.ds(sid * NP, NP)])
  plsc.subcore_barrier()

  for t in range(NS):
    pltpu.sync_copy(degparts.at[pl.ds(t * NP + sid * STR, STR)],
                    sbuf.at[pl.ds(t * STR, STR)])

  def red(j, _):
    v = sbuf[pl.ds(j * L, L)]
    for t in range(1, NS):
      v = v + sbuf[pl.ds(t * STR + j * L, L)]
    dbuf[pl.ds(j * L, L)] = _rsqrt16(v + 1.0)
    return 0
  lax.fori_loop(0, STR // L, red, 0)

  pltpu.sync_copy(dbuf, dinv_sh.at[pl.ds(sid * STR, STR)])
  plsc.subcore_barrier()

  pltpu.sync_copy(dinv_sh, deg)
  pltpu.sync_copy(hp_hbm.at[0], h0d)
  pltpu.sync_copy(hp_hbm.at[1], h1d)

  def scale(i, _):
    sl = pl.ds(i * L, L)
    dv = deg[sl]
    h0d[sl] = h0d[sl] * dv
    h1d[sl] = h1d[sl] * dv
    acc0[sl] = jnp.zeros((L,), jnp.float32)
    acc1[sl] = jnp.zeros((L,), jnp.float32)
    return 0
  lax.fori_loop(0, N // L, scale, 0)

  pltpu.sync_copy(src_hbm.at[pl.ds(wid * E2, E2)], srcv)
  pltpu.sync_copy(dst_hbm.at[pl.ds(wid * E2, E2)], dstv.at[pl.ds(0, E2)])
  pltpu.sync_copy(ew_hbm.at[pl.ds(wid * E2, E2)], ewv.at[pl.ds(0, E2)])

  def p2(i, _):
    s = srcv[pl.ds(i * L, L)]
    d = dstv[pl.ds(i * L, L)]
    w = ewv[pl.ds(i * L, L)]
    m0 = plsc.load_gather(h0d, [s]) * w
    m1 = plsc.load_gather(h1d, [s]) * w
    plsc.addupdate_scatter(acc0, [d], m0)
    plsc.addupdate_scatter(acc1, [d], m1)
    return 0
  lax.fori_loop(0, E2 // L, p2, 0)

  pltpu.sync_copy(acc0, accp0_hbm.at[wid])
  pltpu.sync_copy(acc1, accp1_hbm.at[wid])

  @pl.when(wid == 0)
  def _():
    pltpu.sync_copy(deg.at[pl.ds(0, N)], dinv_hbm)


@functools.partial(
    pl.kernel,
    out_type=(
        jax.ShapeDtypeStruct((NW, N), jnp.float32),
        jax.ShapeDtypeStruct((NW, N), jnp.float32),
        jax.ShapeDtypeStruct((N,), jnp.float32),
    ),
    mesh=_sc_mesh,
    scratch_types=[
        pltpu.VMEM((E1,), jnp.int32),
        pltpu.VMEM((E1,), jnp.float32),
        pltpu.VMEM((E2,), jnp.int32),
        pltpu.VMEM((NP,), jnp.float32),
        pltpu.VMEM((NP,), jnp.float32),
        pltpu.VMEM((STR,), jnp.float32),
        pltpu.VMEM((N,), jnp.float32),
        pltpu.VMEM((N,), jnp.float32),
        pltpu.VMEM((N,), jnp.float32),
        pltpu.VMEM((N,), jnp.float32),
        pltpu.VMEM_SHARED((NS * NP,), jnp.float32),
        pltpu.VMEM_SHARED((NP,), jnp.float32),
    ],
    compiler_params=pltpu.CompilerParams(needs_layout_passes=False),
)
def _sc_main(*refs):
  _sc_main_body(*refs)


def _tc_mm_body(w1t_ref, x_ref, hp_ref):
  hp_ref[...] = lax.dot_general(
      w1t_ref[...], x_ref[...], (((1,), (1,)), ((), ())),
      preferred_element_type=jnp.float32)


def _tc_mm(w1t, x):
  return pl.pallas_call(
      _tc_mm_body,
      out_shape=jax.ShapeDtypeStruct((2, N), jnp.float32),
  )(w1t, x)


def _tc_final_body(accp0_ref, accp1_ref, h0_ref, h1_ref, dinv_ref,
                   wfc0_ref, wfc1_ref, b1_ref, bfc_ref, o_ref):
  dv = dinv_ref[...]
  a0 = jnp.sum(accp0_ref[...], axis=0)
  a1 = jnp.sum(accp1_ref[...], axis=0)
  o0 = jnp.maximum(dv * (a0 + h0_ref[...] * dv) + b1_ref[0], 0.0)
  o1 = jnp.maximum(dv * (a1 + h1_ref[...] * dv) + b1_ref[1], 0.0)
  logit = (jnp.sum(o0 * wfc0_ref[...]) + jnp.sum(o1 * wfc1_ref[...])
           + bfc_ref[0])
  o_ref[0, 0] = 1.0 / (1.0 + jnp.exp(-logit))


def _tc_final(accp0, accp1, h0, h1, dinv, wfc0, wfc1, b1, bfc):
  vm = pl.BlockSpec(memory_space=pltpu.VMEM)
  sm = pl.BlockSpec(memory_space=pltpu.SMEM)
  return pl.pallas_call(
      _tc_final_body,
      in_specs=[vm, vm, vm, vm, vm, vm, vm, sm, sm],
      out_specs=sm,
      out_shape=jax.ShapeDtypeStruct((1, 1), jnp.float32),
  )(accp0, accp1, h0, h1, dinv, wfc0, wfc1, b1, bfc)


def kernel(x, edge_list, edge_attr, W1, b1, Wfc, bfc):
  src = edge_list[0]
  dst = edge_list[1]
  ew = edge_attr.astype(jnp.float32)

  hp = _tc_mm(W1.T, x)
  accp0, accp1, dinv = _sc_main(src, dst, ew, hp)

  wfcp = Wfc.reshape(N, 2).T
  out = _tc_final(
      accp0.reshape(NW, 80, 125),
      accp1.reshape(NW, 80, 125),
      hp[0].reshape(80, 125),
      hp[1].reshape(80, 125),
      dinv.reshape(80, 125),
      wfcp[0].reshape(80, 125),
      wfcp[1].reshape(80, 125),
      b1,
      bfc,
  )
  return out.reshape(())
